# Initial kernel scaffold; baseline (speedup 1.0000x reference)
import jax
import jax.numpy as jnp
from jax.experimental import pallas as pl


def kernel(x, edge_index, W1, b1, W2, b2, W3, b3):
    raise NotImplementedError("write your pallas kernel here")


# speedup vs baseline: 2.8046x; 2.8046x over previous
import functools

import jax
import jax.numpy as jnp
from jax import lax
from jax.experimental import pallas as pl
from jax.experimental.pallas import tpu as pltpu
from jax.experimental.pallas import tpu_sc as plsc

_N = 10000
_NPAD = 10240
_E = 320000
_CH = 128
_K = 80
_EPAD = 2 * 16 * _K * _CH
_RPT = _NPAD // 16
_ZR = 160
_W = 32

_mesh = plsc.VectorSubcoreMesh(core_axis_name="c", subcore_axis_name="s")


@functools.partial(
    pl.kernel,
    out_type=jax.ShapeDtypeStruct((2, _NPAD, _W), jnp.float32),
    mesh=_mesh,
    compiler_params=pltpu.CompilerParams(use_tc_tiling_on_sc=False),
    scratch_types=[
        pltpu.VMEM((_K, _CH), jnp.int32),
        pltpu.VMEM((_K, _CH), jnp.int32),
        pltpu.VMEM((_CH, _W), jnp.float32),
        pltpu.VMEM((_ZR, _W), jnp.float32),
        pltpu.VMEM_SHARED((_NPAD, _W), jnp.float32),
        pltpu.SemaphoreType.DMA,
    ],
)
def _prop(h_hbm, src_hbm, dst_hbm, zeros_hbm, out_hbm,
          src_v, dst_v, rows_v, zbuf_v, agg_sh, sem):
    c = lax.axis_index("c")
    s = lax.axis_index("s")
    pltpu.sync_copy(src_hbm.at[c, s], src_v)
    pltpu.sync_copy(dst_hbm.at[c, s], dst_v)
    pltpu.sync_copy(zeros_hbm, zbuf_v)
    base = s * _RPT
    for z in range(_RPT // _ZR):
        pltpu.sync_copy(zbuf_v, agg_sh.at[pl.ds(base + z * _ZR, _ZR)])
    plsc.subcore_barrier()

    def body(j, carry):
        pltpu.async_copy(h_hbm.at[src_v.at[j]], rows_v, sem).wait()
        pltpu.sync_copy(rows_v, agg_sh.at[dst_v.at[j]], add=True)
        return carry

    lax.fori_loop(0, _K, body, 0)
    plsc.subcore_barrier()
    for z in range(_RPT // _ZR):
        pltpu.sync_copy(agg_sh.at[pl.ds(base + z * _ZR, _ZR)], zbuf_v)
        pltpu.sync_copy(zbuf_v, out_hbm.at[c, pl.ds(base + z * _ZR, _ZR)])


@functools.partial(
    pl.kernel,
    out_type=jax.ShapeDtypeStruct((2, _NPAD, 16), jnp.float32),
    mesh=_mesh,
    compiler_params=pltpu.CompilerParams(use_tc_tiling_on_sc=False),
    scratch_types=[
        pltpu.VMEM((_K, _CH), jnp.int32),
        pltpu.VMEM((_CH, 16), jnp.float32),
        pltpu.VMEM((_ZR, 16), jnp.float32),
        pltpu.VMEM_SHARED((_NPAD, 16), jnp.float32),
    ],
)
def _deg_half(idx_hbm, ones_hbm, zeros_hbm, out_hbm,
              idx_v, ones_v, buf_v, deg_sh):
    c = lax.axis_index("c")
    s = lax.axis_index("s")
    pltpu.sync_copy(idx_hbm.at[c, s], idx_v)
    pltpu.sync_copy(ones_hbm, ones_v)
    pltpu.sync_copy(zeros_hbm, buf_v)
    base = s * _RPT
    for z in range(_RPT // _ZR):
        pltpu.sync_copy(buf_v, deg_sh.at[pl.ds(base + z * _ZR, _ZR)])
    plsc.subcore_barrier()

    def body(j, carry):
        pltpu.sync_copy(ones_v, deg_sh.at[idx_v.at[j]], add=True)
        return carry

    lax.fori_loop(0, _K, body, 0)
    plsc.subcore_barrier()
    for z in range(_RPT // _ZR):
        pltpu.sync_copy(deg_sh.at[pl.ds(base + z * _ZR, _ZR)], buf_v)
        pltpu.sync_copy(buf_v, out_hbm.at[c, pl.ds(base + z * _ZR, _ZR)])


_BLK = 256
_G = _NPAD // _BLK

_HI = lax.Precision.HIGHEST
_NS = 128 // _W


def _norms_body(dega_ref, degb_ref, x_ref, h0_ref, h1_ref, h2_ref, h3_ref,
                ns_ref, nd_ref):
    dsrc = dega_ref[0] + dega_ref[1]
    ddst = degb_ref[0] + degb_ref[1]
    ns = jnp.where(dsrc > 0, lax.rsqrt(dsrc), 0.0)
    nd = jnp.where(ddst > 0, lax.rsqrt(ddst), 0.0)
    h = x_ref[...] * ns[:, 0:1]
    for i, r in enumerate((h0_ref, h1_ref, h2_ref, h3_ref)):
        r[...] = h[:, i * _W:(i + 1) * _W]
    ns_ref[...] = ns[:, 0:8]
    nd_ref[...] = nd[:, 0:8]


_norms_call = pl.pallas_call(
    _norms_body,
    grid=(_G,),
    in_specs=[
        pl.BlockSpec((2, _BLK, 16), lambda i: (0, i, 0)),
        pl.BlockSpec((2, _BLK, 16), lambda i: (0, i, 0)),
        pl.BlockSpec((_BLK, 128), lambda i: (i, 0)),
    ],
    out_specs=[pl.BlockSpec((_BLK, _W), lambda i: (i, 0)) for _ in range(_NS)]
    + [pl.BlockSpec((_BLK, 8), lambda i: (i, 0)) for _ in range(2)],
    out_shape=[jax.ShapeDtypeStruct((_NPAD, _W), jnp.float32) for _ in range(_NS)]
    + [jax.ShapeDtypeStruct((_NPAD, 8), jnp.float32) for _ in range(2)],
)


def _agg_full(part_refs, nd):
    cols = [(p[0] + p[1]) * nd for p in part_refs]
    return jnp.concatenate(cols, axis=1)


def _layer_body(p0, p1, p2, p3, ns_ref, nd_ref, w_ref, b_ref,
                h0_ref, h1_ref, h2_ref, h3_ref):
    t = _agg_full((p0, p1, p2, p3), nd_ref[:, 0:1])
    y = jnp.dot(t, w_ref[...], preferred_element_type=jnp.float32,
                precision=_HI) + b_ref[...]
    h = jnp.maximum(y, 0.0) * ns_ref[:, 0:1]
    for i, r in enumerate((h0_ref, h1_ref, h2_ref, h3_ref)):
        r[...] = h[:, i * _W:(i + 1) * _W]


_layer_call = pl.pallas_call(
    _layer_body,
    grid=(_G,),
    in_specs=[pl.BlockSpec((2, _BLK, _W), lambda i: (0, i, 0)) for _ in range(_NS)]
    + [
        pl.BlockSpec((_BLK, 8), lambda i: (i, 0)),
        pl.BlockSpec((_BLK, 8), lambda i: (i, 0)),
        pl.BlockSpec((128, 128), lambda i: (0, 0)),
        pl.BlockSpec((1, 128), lambda i: (0, 0)),
    ],
    out_specs=[pl.BlockSpec((_BLK, _W), lambda i: (i, 0)) for _ in range(_NS)],
    out_shape=[jax.ShapeDtypeStruct((_NPAD, _W), jnp.float32) for _ in range(_NS)],
)


def _layer2_body(p0, p1, p2, p3, ns_ref, nd_ref, w_ref, b_ref, wp_ref,
                 t0_ref, t1_ref):
    t = _agg_full((p0, p1, p2, p3), nd_ref[:, 0:1])
    y = jnp.dot(t, w_ref[...], preferred_element_type=jnp.float32,
                precision=_HI) + b_ref[...]
    h = jnp.maximum(y, 0.0) * ns_ref[:, 0:1]
    o = jnp.dot(h, wp_ref[...], preferred_element_type=jnp.float32,
                precision=_HI)
    t0_ref[...] = o[:, :_W]
    t1_ref[...] = o[:, _W:]


_layer2_call = pl.pallas_call(
    _layer2_body,
    grid=(_G,),
    in_specs=[pl.BlockSpec((2, _BLK, _W), lambda i: (0, i, 0)) for _ in range(_NS)]
    + [
        pl.BlockSpec((_BLK, 8), lambda i: (i, 0)),
        pl.BlockSpec((_BLK, 8), lambda i: (i, 0)),
        pl.BlockSpec((128, 128), lambda i: (0, 0)),
        pl.BlockSpec((1, 128), lambda i: (0, 0)),
        pl.BlockSpec((128, 64), lambda i: (0, 0)),
    ],
    out_specs=[pl.BlockSpec((_BLK, _W), lambda i: (i, 0)) for _ in range(2)],
    out_shape=[jax.ShapeDtypeStruct((_NPAD, _W), jnp.float32) for _ in range(2)],
)


def _final_body(p0, p1, nd_ref, b_ref, o_ref):
    o_ref[...] = _agg_full((p0, p1), nd_ref[:, 0:1]) + b_ref[...]


_final_call = pl.pallas_call(
    _final_body,
    grid=(_G,),
    in_specs=[pl.BlockSpec((2, _BLK, _W), lambda i: (0, i, 0)) for _ in range(2)]
    + [
        pl.BlockSpec((_BLK, 8), lambda i: (i, 0)),
        pl.BlockSpec((1, 64), lambda i: (0, 0)),
    ],
    out_specs=pl.BlockSpec((_BLK, 64), lambda i: (i, 0)),
    out_shape=jax.ShapeDtypeStruct((_NPAD, 64), jnp.float32),
)


def kernel(x, edge_index, W1, b1, W2, b2, W3, b3):
    f32 = jnp.float32
    src = edge_index[0].astype(jnp.int32)
    dst = edge_index[1].astype(jnp.int32)
    pad = jnp.full((_EPAD - _E,), _NPAD - 1, jnp.int32)
    srcp = jnp.concatenate([src, pad]).reshape(2, 16, _K, _CH)
    dstp = jnp.concatenate([dst, pad]).reshape(2, 16, _K, _CH)
    xpad = jnp.pad(x, ((0, _NPAD - _N), (0, 0)))
    ones16 = jnp.ones((_CH, 16), f32)
    zeros16 = jnp.zeros((_ZR, 16), f32)
    zerosw = jnp.zeros((_ZR, _W), f32)
    w3p = jnp.pad(W3, ((0, 0), (0, 64 - 40)))
    b3p = jnp.pad(b3, (0, 64 - 40)).reshape(1, 64)

    dega = _deg_half(srcp, ones16, zeros16)
    degb = _deg_half(dstp, ones16, zeros16)
    outs = _norms_call(dega, degb, xpad)
    h0s, ns, nd = outs[:_NS], outs[_NS], outs[_NS + 1]
    p1 = [_prop(h, srcp, dstp, zerosw) for h in h0s]
    h1s = _layer_call(*p1, ns, nd, W1, b1.reshape(1, 128))
    p2 = [_prop(h, srcp, dstp, zerosw) for h in h1s]
    t3 = _layer2_call(*p2, ns, nd, W2, b2.reshape(1, 128), w3p)
    p3 = [_prop(t, srcp, dstp, zerosw) for t in t3]
    out = _final_call(*p3, nd, b3p)
    return out[:_N, :40]


# speedup vs baseline: 3.5743x; 1.2744x over previous
import functools

import jax
import jax.numpy as jnp
from jax import lax
from jax.experimental import pallas as pl
from jax.experimental.pallas import tpu as pltpu
from jax.experimental.pallas import tpu_sc as plsc

_N = 10000
_NPAD = 10240
_E = 320000
_CH = 128
_K = 80
_EPAD = 2 * 16 * _K * _CH
_RPT = _NPAD // 16
_ZR = 160
_W = 32

_mesh = plsc.VectorSubcoreMesh(core_axis_name="c", subcore_axis_name="s")


@functools.partial(
    pl.kernel,
    out_type=jax.ShapeDtypeStruct((2, _NPAD, _W), jnp.float32),
    mesh=_mesh,
    compiler_params=pltpu.CompilerParams(use_tc_tiling_on_sc=False),
    scratch_types=[
        pltpu.VMEM((_K, _CH), jnp.int32),
        pltpu.VMEM((_K, _CH), jnp.int32),
        pltpu.VMEM((2, _CH, _W), jnp.float32),
        pltpu.VMEM((_ZR, _W), jnp.float32),
        pltpu.VMEM_SHARED((_NPAD, _W), jnp.float32),
        pltpu.SemaphoreType.DMA,
    ],
)
def _prop(h_hbm, src_hbm, dst_hbm, zeros_hbm, out_hbm,
          src_v, dst_v, rows_v, zbuf_v, agg_sh, sem):
    c = lax.axis_index("c")
    s = lax.axis_index("s")
    pltpu.sync_copy(src_hbm.at[c, s], src_v)
    pltpu.sync_copy(dst_hbm.at[c, s], dst_v)
    pltpu.sync_copy(zeros_hbm, zbuf_v)
    base = s * _RPT
    for z in range(_RPT // _ZR):
        pltpu.sync_copy(zbuf_v, agg_sh.at[pl.ds(base + z * _ZR, _ZR)])
    plsc.subcore_barrier()

    pltpu.async_copy(h_hbm.at[src_v.at[0]], rows_v.at[0], sem)

    def body(j, carry):
        p = lax.rem(j, 2)
        pn = lax.rem(j + 1, 2)

        @pl.when(j + 1 < _K)
        def _():
            pltpu.async_copy(h_hbm.at[src_v.at[j + 1]], rows_v.at[pn], sem)

        pltpu.make_async_copy(h_hbm.at[src_v.at[j]], rows_v.at[p], sem).wait()
        pltpu.sync_copy(rows_v.at[p], agg_sh.at[dst_v.at[j]], add=True)
        return carry

    lax.fori_loop(0, _K, body, 0)
    plsc.subcore_barrier()
    for z in range(_RPT // _ZR):
        pltpu.sync_copy(agg_sh.at[pl.ds(base + z * _ZR, _ZR)], zbuf_v)
        pltpu.sync_copy(zbuf_v, out_hbm.at[c, pl.ds(base + z * _ZR, _ZR)])


@functools.partial(
    pl.kernel,
    out_type=jax.ShapeDtypeStruct((2, _NPAD, 16), jnp.float32),
    mesh=_mesh,
    compiler_params=pltpu.CompilerParams(use_tc_tiling_on_sc=False),
    scratch_types=[
        pltpu.VMEM((_K, _CH), jnp.int32),
        pltpu.VMEM((_CH, 16), jnp.float32),
        pltpu.VMEM((_ZR, 16), jnp.float32),
        pltpu.VMEM_SHARED((_NPAD, 16), jnp.float32),
    ],
)
def _deg_half(idx_hbm, ones_hbm, zeros_hbm, out_hbm,
              idx_v, ones_v, buf_v, deg_sh):
    c = lax.axis_index("c")
    s = lax.axis_index("s")
    pltpu.sync_copy(idx_hbm.at[c, s], idx_v)
    pltpu.sync_copy(ones_hbm, ones_v)
    pltpu.sync_copy(zeros_hbm, buf_v)
    base = s * _RPT
    for z in range(_RPT // _ZR):
        pltpu.sync_copy(buf_v, deg_sh.at[pl.ds(base + z * _ZR, _ZR)])
    plsc.subcore_barrier()

    def body(j, carry):
        pltpu.sync_copy(ones_v, deg_sh.at[idx_v.at[j]], add=True)
        return carry

    lax.fori_loop(0, _K, body, 0)
    plsc.subcore_barrier()
    for z in range(_RPT // _ZR):
        pltpu.sync_copy(deg_sh.at[pl.ds(base + z * _ZR, _ZR)], buf_v)
        pltpu.sync_copy(buf_v, out_hbm.at[c, pl.ds(base + z * _ZR, _ZR)])


_BLK = 256
_G = _NPAD // _BLK

_HI = lax.Precision.HIGHEST
_NS = 128 // _W


def _norms_body(dega_ref, degb_ref, x_ref, h0_ref, h1_ref, h2_ref, h3_ref,
                ns_ref, nd_ref):
    dsrc = dega_ref[0] + dega_ref[1]
    ddst = degb_ref[0] + degb_ref[1]
    ns = jnp.where(dsrc > 0, lax.rsqrt(dsrc), 0.0)
    nd = jnp.where(ddst > 0, lax.rsqrt(ddst), 0.0)
    h = x_ref[...] * ns[:, 0:1]
    for i, r in enumerate((h0_ref, h1_ref, h2_ref, h3_ref)):
        r[...] = h[:, i * _W:(i + 1) * _W]
    ns_ref[...] = ns[:, 0:8]
    nd_ref[...] = nd[:, 0:8]


_norms_call = pl.pallas_call(
    _norms_body,
    grid=(_G,),
    in_specs=[
        pl.BlockSpec((2, _BLK, 16), lambda i: (0, i, 0)),
        pl.BlockSpec((2, _BLK, 16), lambda i: (0, i, 0)),
        pl.BlockSpec((_BLK, 128), lambda i: (i, 0)),
    ],
    out_specs=[pl.BlockSpec((_BLK, _W), lambda i: (i, 0)) for _ in range(_NS)]
    + [pl.BlockSpec((_BLK, 8), lambda i: (i, 0)) for _ in range(2)],
    out_shape=[jax.ShapeDtypeStruct((_NPAD, _W), jnp.float32) for _ in range(_NS)]
    + [jax.ShapeDtypeStruct((_NPAD, 8), jnp.float32) for _ in range(2)],
)


def _agg_full(part_refs, nd):
    cols = [(p[0] + p[1]) * nd for p in part_refs]
    return jnp.concatenate(cols, axis=1)


def _layer_body(p0, p1, p2, p3, ns_ref, nd_ref, w_ref, b_ref,
                h0_ref, h1_ref, h2_ref, h3_ref):
    t = _agg_full((p0, p1, p2, p3), nd_ref[:, 0:1])
    y = jnp.dot(t, w_ref[...], preferred_element_type=jnp.float32,
                precision=_HI) + b_ref[...]
    h = jnp.maximum(y, 0.0) * ns_ref[:, 0:1]
    for i, r in enumerate((h0_ref, h1_ref, h2_ref, h3_ref)):
        r[...] = h[:, i * _W:(i + 1) * _W]


_layer_call = pl.pallas_call(
    _layer_body,
    grid=(_G,),
    in_specs=[pl.BlockSpec((2, _BLK, _W), lambda i: (0, i, 0)) for _ in range(_NS)]
    + [
        pl.BlockSpec((_BLK, 8), lambda i: (i, 0)),
        pl.BlockSpec((_BLK, 8), lambda i: (i, 0)),
        pl.BlockSpec((128, 128), lambda i: (0, 0)),
        pl.BlockSpec((1, 128), lambda i: (0, 0)),
    ],
    out_specs=[pl.BlockSpec((_BLK, _W), lambda i: (i, 0)) for _ in range(_NS)],
    out_shape=[jax.ShapeDtypeStruct((_NPAD, _W), jnp.float32) for _ in range(_NS)],
)


def _layer2_body(p0, p1, p2, p3, ns_ref, nd_ref, w_ref, b_ref, wp_ref,
                 t0_ref, t1_ref):
    t = _agg_full((p0, p1, p2, p3), nd_ref[:, 0:1])
    y = jnp.dot(t, w_ref[...], preferred_element_type=jnp.float32,
                precision=_HI) + b_ref[...]
    h = jnp.maximum(y, 0.0) * ns_ref[:, 0:1]
    o = jnp.dot(h, wp_ref[...], preferred_element_type=jnp.float32,
                precision=_HI)
    t0_ref[...] = o[:, :_W]
    t1_ref[...] = o[:, _W:]


_layer2_call = pl.pallas_call(
    _layer2_body,
    grid=(_G,),
    in_specs=[pl.BlockSpec((2, _BLK, _W), lambda i: (0, i, 0)) for _ in range(_NS)]
    + [
        pl.BlockSpec((_BLK, 8), lambda i: (i, 0)),
        pl.BlockSpec((_BLK, 8), lambda i: (i, 0)),
        pl.BlockSpec((128, 128), lambda i: (0, 0)),
        pl.BlockSpec((1, 128), lambda i: (0, 0)),
        pl.BlockSpec((128, 64), lambda i: (0, 0)),
    ],
    out_specs=[pl.BlockSpec((_BLK, _W), lambda i: (i, 0)) for _ in range(2)],
    out_shape=[jax.ShapeDtypeStruct((_NPAD, _W), jnp.float32) for _ in range(2)],
)


def _final_body(p0, p1, nd_ref, b_ref, o_ref):
    o_ref[...] = _agg_full((p0, p1), nd_ref[:, 0:1]) + b_ref[...]


_final_call = pl.pallas_call(
    _final_body,
    grid=(_G,),
    in_specs=[pl.BlockSpec((2, _BLK, _W), lambda i: (0, i, 0)) for _ in range(2)]
    + [
        pl.BlockSpec((_BLK, 8), lambda i: (i, 0)),
        pl.BlockSpec((1, 64), lambda i: (0, 0)),
    ],
    out_specs=pl.BlockSpec((_BLK, 64), lambda i: (i, 0)),
    out_shape=jax.ShapeDtypeStruct((_NPAD, 64), jnp.float32),
)


def kernel(x, edge_index, W1, b1, W2, b2, W3, b3):
    f32 = jnp.float32
    src = edge_index[0].astype(jnp.int32)
    dst = edge_index[1].astype(jnp.int32)
    pad = jnp.full((_EPAD - _E,), _NPAD - 1, jnp.int32)
    srcp = jnp.concatenate([src, pad]).reshape(2, 16, _K, _CH)
    dstp = jnp.concatenate([dst, pad]).reshape(2, 16, _K, _CH)
    xpad = jnp.pad(x, ((0, _NPAD - _N), (0, 0)))
    ones16 = jnp.ones((_CH, 16), f32)
    zeros16 = jnp.zeros((_ZR, 16), f32)
    zerosw = jnp.zeros((_ZR, _W), f32)
    w3p = jnp.pad(W3, ((0, 0), (0, 64 - 40)))
    b3p = jnp.pad(b3, (0, 64 - 40)).reshape(1, 64)

    dega = _deg_half(srcp, ones16, zeros16)
    degb = _deg_half(dstp, ones16, zeros16)
    outs = _norms_call(dega, degb, xpad)
    h0s, ns, nd = outs[:_NS], outs[_NS], outs[_NS + 1]
    p1 = [_prop(h, srcp, dstp, zerosw) for h in h0s]
    h1s = _layer_call(*p1, ns, nd, W1, b1.reshape(1, 128))
    p2 = [_prop(h, srcp, dstp, zerosw) for h in h1s]
    t3 = _layer2_call(*p2, ns, nd, W2, b2.reshape(1, 128), w3p)
    p3 = [_prop(t, srcp, dstp, zerosw) for t in t3]
    out = _final_call(*p3, nd, b3p)
    return out[:_N, :40]


# speedup vs baseline: 3.6060x; 1.0089x over previous
import functools

import jax
import jax.numpy as jnp
from jax import lax
from jax.experimental import pallas as pl
from jax.experimental.pallas import tpu as pltpu
from jax.experimental.pallas import tpu_sc as plsc

_N = 10000
_NPAD = 10240
_E = 320000
_CH = 128
_K = 80
_EPAD = 2 * 16 * _K * _CH
_RPT = _NPAD // 16
_ZR = 160
_W = 32
_NBUF = 4

_mesh = plsc.VectorSubcoreMesh(core_axis_name="c", subcore_axis_name="s")


@functools.partial(
    pl.kernel,
    out_type=jax.ShapeDtypeStruct((2, _NPAD, _W), jnp.float32),
    mesh=_mesh,
    compiler_params=pltpu.CompilerParams(use_tc_tiling_on_sc=False),
    scratch_types=[
        pltpu.VMEM((_K, _CH), jnp.int32),
        pltpu.VMEM((_K, _CH), jnp.int32),
        pltpu.VMEM((_NBUF, _CH, _W), jnp.float32),
        pltpu.VMEM((_ZR, _W), jnp.float32),
        pltpu.VMEM_SHARED((_NPAD, _W), jnp.float32),
        pltpu.SemaphoreType.DMA,
    ],
)
def _prop(h_hbm, src_hbm, dst_hbm, zeros_hbm, out_hbm,
          src_v, dst_v, rows_v, zbuf_v, agg_sh, sem):
    c = lax.axis_index("c")
    s = lax.axis_index("s")
    pltpu.sync_copy(src_hbm.at[c, s], src_v)
    pltpu.sync_copy(dst_hbm.at[c, s], dst_v)
    pltpu.sync_copy(zeros_hbm, zbuf_v)
    base = s * _RPT
    for z in range(_RPT // _ZR):
        pltpu.sync_copy(zbuf_v, agg_sh.at[pl.ds(base + z * _ZR, _ZR)])
    plsc.subcore_barrier()

    for j0 in range(_NBUF - 1):
        pltpu.async_copy(h_hbm.at[src_v.at[j0]], rows_v.at[j0], sem)

    def body(j, carry):
        p = lax.rem(j, _NBUF)
        pn = lax.rem(j + _NBUF - 1, _NBUF)

        @pl.when(j + _NBUF - 1 < _K)
        def _():
            pltpu.async_copy(h_hbm.at[src_v.at[j + _NBUF - 1]], rows_v.at[pn], sem)

        pltpu.make_async_copy(h_hbm.at[src_v.at[j]], rows_v.at[p], sem).wait()
        pltpu.sync_copy(rows_v.at[p], agg_sh.at[dst_v.at[j]], add=True)
        return carry

    lax.fori_loop(0, _K, body, 0)
    plsc.subcore_barrier()
    for z in range(_RPT // _ZR):
        pltpu.sync_copy(agg_sh.at[pl.ds(base + z * _ZR, _ZR)], zbuf_v)
        pltpu.sync_copy(zbuf_v, out_hbm.at[c, pl.ds(base + z * _ZR, _ZR)])


@functools.partial(
    pl.kernel,
    out_type=jax.ShapeDtypeStruct((2, _NPAD, 16), jnp.float32),
    mesh=_mesh,
    compiler_params=pltpu.CompilerParams(use_tc_tiling_on_sc=False),
    scratch_types=[
        pltpu.VMEM((_K, _CH), jnp.int32),
        pltpu.VMEM((_CH, 16), jnp.float32),
        pltpu.VMEM((_ZR, 16), jnp.float32),
        pltpu.VMEM_SHARED((_NPAD, 16), jnp.float32),
    ],
)
def _deg_half(idx_hbm, ones_hbm, zeros_hbm, out_hbm,
              idx_v, ones_v, buf_v, deg_sh):
    c = lax.axis_index("c")
    s = lax.axis_index("s")
    pltpu.sync_copy(idx_hbm.at[c, s], idx_v)
    pltpu.sync_copy(ones_hbm, ones_v)
    pltpu.sync_copy(zeros_hbm, buf_v)
    base = s * _RPT
    for z in range(_RPT // _ZR):
        pltpu.sync_copy(buf_v, deg_sh.at[pl.ds(base + z * _ZR, _ZR)])
    plsc.subcore_barrier()

    def body(j, carry):
        pltpu.sync_copy(ones_v, deg_sh.at[idx_v.at[j]], add=True)
        return carry

    lax.fori_loop(0, _K, body, 0)
    plsc.subcore_barrier()
    for z in range(_RPT // _ZR):
        pltpu.sync_copy(deg_sh.at[pl.ds(base + z * _ZR, _ZR)], buf_v)
        pltpu.sync_copy(buf_v, out_hbm.at[c, pl.ds(base + z * _ZR, _ZR)])


_BLK = 256
_G = _NPAD // _BLK

_HI = lax.Precision.HIGHEST
_NS = 128 // _W


def _norms_body(dega_ref, degb_ref, x_ref, h0_ref, h1_ref, h2_ref, h3_ref,
                ns_ref, nd_ref):
    dsrc = dega_ref[0] + dega_ref[1]
    ddst = degb_ref[0] + degb_ref[1]
    ns = jnp.where(dsrc > 0, lax.rsqrt(dsrc), 0.0)
    nd = jnp.where(ddst > 0, lax.rsqrt(ddst), 0.0)
    h = x_ref[...] * ns[:, 0:1]
    for i, r in enumerate((h0_ref, h1_ref, h2_ref, h3_ref)):
        r[...] = h[:, i * _W:(i + 1) * _W]
    ns_ref[...] = ns[:, 0:8]
    nd_ref[...] = nd[:, 0:8]


_norms_call = pl.pallas_call(
    _norms_body,
    grid=(_G,),
    in_specs=[
        pl.BlockSpec((2, _BLK, 16), lambda i: (0, i, 0)),
        pl.BlockSpec((2, _BLK, 16), lambda i: (0, i, 0)),
        pl.BlockSpec((_BLK, 128), lambda i: (i, 0)),
    ],
    out_specs=[pl.BlockSpec((_BLK, _W), lambda i: (i, 0)) for _ in range(_NS)]
    + [pl.BlockSpec((_BLK, 8), lambda i: (i, 0)) for _ in range(2)],
    out_shape=[jax.ShapeDtypeStruct((_NPAD, _W), jnp.float32) for _ in range(_NS)]
    + [jax.ShapeDtypeStruct((_NPAD, 8), jnp.float32) for _ in range(2)],
)


def _agg_full(part_refs, nd):
    cols = [(p[0] + p[1]) * nd for p in part_refs]
    return jnp.concatenate(cols, axis=1)


def _layer_body(p0, p1, p2, p3, ns_ref, nd_ref, w_ref, b_ref,
                h0_ref, h1_ref, h2_ref, h3_ref):
    t = _agg_full((p0, p1, p2, p3), nd_ref[:, 0:1])
    y = jnp.dot(t, w_ref[...], preferred_element_type=jnp.float32,
                precision=_HI) + b_ref[...]
    h = jnp.maximum(y, 0.0) * ns_ref[:, 0:1]
    for i, r in enumerate((h0_ref, h1_ref, h2_ref, h3_ref)):
        r[...] = h[:, i * _W:(i + 1) * _W]


_layer_call = pl.pallas_call(
    _layer_body,
    grid=(_G,),
    in_specs=[pl.BlockSpec((2, _BLK, _W), lambda i: (0, i, 0)) for _ in range(_NS)]
    + [
        pl.BlockSpec((_BLK, 8), lambda i: (i, 0)),
        pl.BlockSpec((_BLK, 8), lambda i: (i, 0)),
        pl.BlockSpec((128, 128), lambda i: (0, 0)),
        pl.BlockSpec((1, 128), lambda i: (0, 0)),
    ],
    out_specs=[pl.BlockSpec((_BLK, _W), lambda i: (i, 0)) for _ in range(_NS)],
    out_shape=[jax.ShapeDtypeStruct((_NPAD, _W), jnp.float32) for _ in range(_NS)],
)


def _layer2_body(p0, p1, p2, p3, ns_ref, nd_ref, w_ref, b_ref, wp_ref,
                 t0_ref, t1_ref):
    t = _agg_full((p0, p1, p2, p3), nd_ref[:, 0:1])
    y = jnp.dot(t, w_ref[...], preferred_element_type=jnp.float32,
                precision=_HI) + b_ref[...]
    h = jnp.maximum(y, 0.0) * ns_ref[:, 0:1]
    o = jnp.dot(h, wp_ref[...], preferred_element_type=jnp.float32,
                precision=_HI)
    t0_ref[...] = o[:, :_W]
    t1_ref[...] = o[:, _W:]


_layer2_call = pl.pallas_call(
    _layer2_body,
    grid=(_G,),
    in_specs=[pl.BlockSpec((2, _BLK, _W), lambda i: (0, i, 0)) for _ in range(_NS)]
    + [
        pl.BlockSpec((_BLK, 8), lambda i: (i, 0)),
        pl.BlockSpec((_BLK, 8), lambda i: (i, 0)),
        pl.BlockSpec((128, 128), lambda i: (0, 0)),
        pl.BlockSpec((1, 128), lambda i: (0, 0)),
        pl.BlockSpec((128, 64), lambda i: (0, 0)),
    ],
    out_specs=[pl.BlockSpec((_BLK, _W), lambda i: (i, 0)) for _ in range(2)],
    out_shape=[jax.ShapeDtypeStruct((_NPAD, _W), jnp.float32) for _ in range(2)],
)


def _final_body(p0, p1, nd_ref, b_ref, o_ref):
    o_ref[...] = _agg_full((p0, p1), nd_ref[:, 0:1]) + b_ref[...]


_final_call = pl.pallas_call(
    _final_body,
    grid=(_G,),
    in_specs=[pl.BlockSpec((2, _BLK, _W), lambda i: (0, i, 0)) for _ in range(2)]
    + [
        pl.BlockSpec((_BLK, 8), lambda i: (i, 0)),
        pl.BlockSpec((1, 64), lambda i: (0, 0)),
    ],
    out_specs=pl.BlockSpec((_BLK, 64), lambda i: (i, 0)),
    out_shape=jax.ShapeDtypeStruct((_NPAD, 64), jnp.float32),
)


def kernel(x, edge_index, W1, b1, W2, b2, W3, b3):
    f32 = jnp.float32
    src = edge_index[0].astype(jnp.int32)
    dst = edge_index[1].astype(jnp.int32)
    pad = jnp.full((_EPAD - _E,), _NPAD - 1, jnp.int32)
    srcp = jnp.concatenate([src, pad]).reshape(2, 16, _K, _CH)
    dstp = jnp.concatenate([dst, pad]).reshape(2, 16, _K, _CH)
    xpad = jnp.pad(x, ((0, _NPAD - _N), (0, 0)))
    ones16 = jnp.ones((_CH, 16), f32)
    zeros16 = jnp.zeros((_ZR, 16), f32)
    zerosw = jnp.zeros((_ZR, _W), f32)
    w3p = jnp.pad(W3, ((0, 0), (0, 64 - 40)))
    b3p = jnp.pad(b3, (0, 64 - 40)).reshape(1, 64)

    dega = _deg_half(srcp, ones16, zeros16)
    degb = _deg_half(dstp, ones16, zeros16)
    outs = _norms_call(dega, degb, xpad)
    h0s, ns, nd = outs[:_NS], outs[_NS], outs[_NS + 1]
    p1 = [_prop(h, srcp, dstp, zerosw) for h in h0s]
    h1s = _layer_call(*p1, ns, nd, W1, b1.reshape(1, 128))
    p2 = [_prop(h, srcp, dstp, zerosw) for h in h1s]
    t3 = _layer2_call(*p2, ns, nd, W2, b2.reshape(1, 128), w3p)
    p3 = [_prop(t, srcp, dstp, zerosw) for t in t3]
    out = _final_call(*p3, nd, b3p)
    return out[:_N, :40]


# speedup vs baseline: 3.7470x; 1.0391x over previous
import functools

import jax
import jax.numpy as jnp
from jax import lax
from jax.experimental import pallas as pl
from jax.experimental.pallas import tpu as pltpu
from jax.experimental.pallas import tpu_sc as plsc

_N = 10000
_NPAD = 10240
_E = 320000
_CH = 128
_K = 80
_EPAD = 2 * 16 * _K * _CH
_RPT = _NPAD // 16
_ZR = 160
_W = 32
_NBUF = 4

_mesh = plsc.VectorSubcoreMesh(core_axis_name="c", subcore_axis_name="s")


def _make_prop(n_slices):
    @functools.partial(
        pl.kernel,
        out_type=jax.ShapeDtypeStruct((2, n_slices, _NPAD, _W), jnp.float32),
        mesh=_mesh,
        compiler_params=pltpu.CompilerParams(use_tc_tiling_on_sc=False),
        scratch_types=[
            pltpu.VMEM((_K, _CH), jnp.int32),
            pltpu.VMEM((_K, _CH), jnp.int32),
            pltpu.VMEM((_NBUF, _CH, _W), jnp.float32),
            pltpu.VMEM((_ZR, _W), jnp.float32),
            pltpu.VMEM((_ZR, _W), jnp.float32),
            pltpu.VMEM_SHARED((_NPAD, _W), jnp.float32),
            pltpu.SemaphoreType.DMA,
        ],
    )
    def _prop_fused(*refs):
        hs = refs[:n_slices]
        (src_hbm, dst_hbm, zeros_hbm, out_hbm,
         src_v, dst_v, rows_v, zbuf_v, obuf_v, agg_sh, sem) = refs[n_slices:]
        c = lax.axis_index("c")
        s = lax.axis_index("s")
        pltpu.sync_copy(src_hbm.at[c, s], src_v)
        pltpu.sync_copy(dst_hbm.at[c, s], dst_v)
        pltpu.sync_copy(zeros_hbm, zbuf_v)
        base = s * _RPT
        for sl in range(n_slices):
            h_hbm = hs[sl]
            for z in range(_RPT // _ZR):
                pltpu.sync_copy(zbuf_v, agg_sh.at[pl.ds(base + z * _ZR, _ZR)])
            plsc.subcore_barrier()

            for j0 in range(_NBUF - 1):
                pltpu.async_copy(h_hbm.at[src_v.at[j0]], rows_v.at[j0], sem)

            def body(j, carry, h_hbm=h_hbm):
                p = lax.rem(j, _NBUF)
                pn = lax.rem(j + _NBUF - 1, _NBUF)

                @pl.when(j + _NBUF - 1 < _K)
                def _():
                    pltpu.async_copy(
                        h_hbm.at[src_v.at[j + _NBUF - 1]], rows_v.at[pn], sem)

                pltpu.make_async_copy(
                    h_hbm.at[src_v.at[j]], rows_v.at[p], sem).wait()
                pltpu.sync_copy(rows_v.at[p], agg_sh.at[dst_v.at[j]], add=True)
                return carry

            lax.fori_loop(0, _K, body, 0)
            plsc.subcore_barrier()
            for z in range(_RPT // _ZR):
                pltpu.sync_copy(agg_sh.at[pl.ds(base + z * _ZR, _ZR)], obuf_v)
                pltpu.sync_copy(obuf_v, out_hbm.at[c, sl, pl.ds(base + z * _ZR, _ZR)])

    return _prop_fused


_prop4 = _make_prop(4)
_prop2 = _make_prop(2)


@functools.partial(
    pl.kernel,
    out_type=jax.ShapeDtypeStruct((2, 2, _NPAD, 16), jnp.float32),
    mesh=_mesh,
    compiler_params=pltpu.CompilerParams(use_tc_tiling_on_sc=False),
    scratch_types=[
        pltpu.VMEM((_K, _CH), jnp.int32),
        pltpu.VMEM((_K, _CH), jnp.int32),
        pltpu.VMEM((_CH, 16), jnp.float32),
        pltpu.VMEM((_ZR, 16), jnp.float32),
        pltpu.VMEM_SHARED((_NPAD, 16), jnp.float32),
        pltpu.VMEM_SHARED((_NPAD, 16), jnp.float32),
    ],
)
def _deg(src_hbm, dst_hbm, ones_hbm, zeros_hbm, out_hbm,
         src_v, dst_v, ones_v, buf_v, dega_sh, degb_sh):
    c = lax.axis_index("c")
    s = lax.axis_index("s")
    pltpu.sync_copy(src_hbm.at[c, s], src_v)
    pltpu.sync_copy(dst_hbm.at[c, s], dst_v)
    pltpu.sync_copy(ones_hbm, ones_v)
    pltpu.sync_copy(zeros_hbm, buf_v)
    base = s * _RPT
    for z in range(_RPT // _ZR):
        pltpu.sync_copy(buf_v, dega_sh.at[pl.ds(base + z * _ZR, _ZR)])
        pltpu.sync_copy(buf_v, degb_sh.at[pl.ds(base + z * _ZR, _ZR)])
    plsc.subcore_barrier()

    def body(j, carry):
        pltpu.sync_copy(ones_v, dega_sh.at[src_v.at[j]], add=True)
        pltpu.sync_copy(ones_v, degb_sh.at[dst_v.at[j]], add=True)
        return carry

    lax.fori_loop(0, _K, body, 0)
    plsc.subcore_barrier()
    for z in range(_RPT // _ZR):
        pltpu.sync_copy(dega_sh.at[pl.ds(base + z * _ZR, _ZR)], buf_v)
        pltpu.sync_copy(buf_v, out_hbm.at[c, 0, pl.ds(base + z * _ZR, _ZR)])
        pltpu.sync_copy(degb_sh.at[pl.ds(base + z * _ZR, _ZR)], buf_v)
        pltpu.sync_copy(buf_v, out_hbm.at[c, 1, pl.ds(base + z * _ZR, _ZR)])


_BLK = 256
_G = _NPAD // _BLK

_HI = lax.Precision.HIGHEST
_NS = 128 // _W


def _norms_body(deg_ref, x_ref, h0_ref, h1_ref, h2_ref, h3_ref,
                ns_ref, nd_ref):
    dsrc = deg_ref[0, 0] + deg_ref[1, 0]
    ddst = deg_ref[0, 1] + deg_ref[1, 1]
    ns = jnp.where(dsrc > 0, lax.rsqrt(dsrc), 0.0)
    nd = jnp.where(ddst > 0, lax.rsqrt(ddst), 0.0)
    h = x_ref[...] * ns[:, 0:1]
    for i, r in enumerate((h0_ref, h1_ref, h2_ref, h3_ref)):
        r[...] = h[:, i * _W:(i + 1) * _W]
    ns_ref[...] = ns[:, 0:8]
    nd_ref[...] = nd[:, 0:8]


_norms_call = pl.pallas_call(
    _norms_body,
    grid=(_G,),
    in_specs=[
        pl.BlockSpec((2, 2, _BLK, 16), lambda i: (0, 0, i, 0)),
        pl.BlockSpec((_BLK, 128), lambda i: (i, 0)),
    ],
    out_specs=[pl.BlockSpec((_BLK, _W), lambda i: (i, 0)) for _ in range(_NS)]
    + [pl.BlockSpec((_BLK, 8), lambda i: (i, 0)) for _ in range(2)],
    out_shape=[jax.ShapeDtypeStruct((_NPAD, _W), jnp.float32) for _ in range(_NS)]
    + [jax.ShapeDtypeStruct((_NPAD, 8), jnp.float32) for _ in range(2)],
)


def _agg_full(p_ref, n_slices, nd):
    cols = [(p_ref[0, k] + p_ref[1, k]) * nd for k in range(n_slices)]
    return jnp.concatenate(cols, axis=1)


def _layer_body(p_ref, ns_ref, nd_ref, w_ref, b_ref,
                h0_ref, h1_ref, h2_ref, h3_ref):
    t = _agg_full(p_ref, _NS, nd_ref[:, 0:1])
    y = jnp.dot(t, w_ref[...], preferred_element_type=jnp.float32,
                precision=_HI) + b_ref[...]
    h = jnp.maximum(y, 0.0) * ns_ref[:, 0:1]
    for i, r in enumerate((h0_ref, h1_ref, h2_ref, h3_ref)):
        r[...] = h[:, i * _W:(i + 1) * _W]


_layer_call = pl.pallas_call(
    _layer_body,
    grid=(_G,),
    in_specs=[
        pl.BlockSpec((2, _NS, _BLK, _W), lambda i: (0, 0, i, 0)),
        pl.BlockSpec((_BLK, 8), lambda i: (i, 0)),
        pl.BlockSpec((_BLK, 8), lambda i: (i, 0)),
        pl.BlockSpec((128, 128), lambda i: (0, 0)),
        pl.BlockSpec((1, 128), lambda i: (0, 0)),
    ],
    out_specs=[pl.BlockSpec((_BLK, _W), lambda i: (i, 0)) for _ in range(_NS)],
    out_shape=[jax.ShapeDtypeStruct((_NPAD, _W), jnp.float32) for _ in range(_NS)],
)


def _layer2_body(p_ref, ns_ref, nd_ref, w_ref, b_ref, wp_ref,
                 t0_ref, t1_ref):
    t = _agg_full(p_ref, _NS, nd_ref[:, 0:1])
    y = jnp.dot(t, w_ref[...], preferred_element_type=jnp.float32,
                precision=_HI) + b_ref[...]
    h = jnp.maximum(y, 0.0) * ns_ref[:, 0:1]
    o = jnp.dot(h, wp_ref[...], preferred_element_type=jnp.float32,
                precision=_HI)
    t0_ref[...] = o[:, :_W]
    t1_ref[...] = o[:, _W:]


_layer2_call = pl.pallas_call(
    _layer2_body,
    grid=(_G,),
    in_specs=[
        pl.BlockSpec((2, _NS, _BLK, _W), lambda i: (0, 0, i, 0)),
        pl.BlockSpec((_BLK, 8), lambda i: (i, 0)),
        pl.BlockSpec((_BLK, 8), lambda i: (i, 0)),
        pl.BlockSpec((128, 128), lambda i: (0, 0)),
        pl.BlockSpec((1, 128), lambda i: (0, 0)),
        pl.BlockSpec((128, 64), lambda i: (0, 0)),
    ],
    out_specs=[pl.BlockSpec((_BLK, _W), lambda i: (i, 0)) for _ in range(2)],
    out_shape=[jax.ShapeDtypeStruct((_NPAD, _W), jnp.float32) for _ in range(2)],
)


def _final_body(p_ref, nd_ref, b_ref, o_ref):
    o_ref[...] = _agg_full(p_ref, 2, nd_ref[:, 0:1]) + b_ref[...]


_final_call = pl.pallas_call(
    _final_body,
    grid=(_G,),
    in_specs=[
        pl.BlockSpec((2, 2, _BLK, _W), lambda i: (0, 0, i, 0)),
        pl.BlockSpec((_BLK, 8), lambda i: (i, 0)),
        pl.BlockSpec((1, 64), lambda i: (0, 0)),
    ],
    out_specs=pl.BlockSpec((_BLK, 64), lambda i: (i, 0)),
    out_shape=jax.ShapeDtypeStruct((_NPAD, 64), jnp.float32),
)


def kernel(x, edge_index, W1, b1, W2, b2, W3, b3):
    f32 = jnp.float32
    src = edge_index[0].astype(jnp.int32)
    dst = edge_index[1].astype(jnp.int32)
    pad = jnp.full((_EPAD - _E,), _NPAD - 1, jnp.int32)
    srcp = jnp.concatenate([src, pad]).reshape(2, 16, _K, _CH)
    dstp = jnp.concatenate([dst, pad]).reshape(2, 16, _K, _CH)
    xpad = jnp.pad(x, ((0, _NPAD - _N), (0, 0)))
    ones16 = jnp.ones((_CH, 16), f32)
    zeros16 = jnp.zeros((_ZR, 16), f32)
    zerosw = jnp.zeros((_ZR, _W), f32)
    w3p = jnp.pad(W3, ((0, 0), (0, 64 - 40)))
    b3p = jnp.pad(b3, (0, 64 - 40)).reshape(1, 64)

    degp = _deg(srcp, dstp, ones16, zeros16)
    outs = _norms_call(degp, xpad)
    h0s, ns, nd = outs[:_NS], outs[_NS], outs[_NS + 1]
    p1 = _prop4(*h0s, srcp, dstp, zerosw)
    h1s = _layer_call(p1, ns, nd, W1, b1.reshape(1, 128))
    p2 = _prop4(*h1s, srcp, dstp, zerosw)
    t3 = _layer2_call(p2, ns, nd, W2, b2.reshape(1, 128), w3p)
    p3 = _prop2(*t3, srcp, dstp, zerosw)
    out = _final_call(p3, nd, b3p)
    return out[:_N, :40]
